# Initial kernel scaffold; baseline (speedup 1.0000x reference)
import functools

import jax
import jax.numpy as jnp
from jax import lax
from jax.experimental import pallas as pl
from jax.experimental.pallas import tpu as pltpu
from jax.experimental.pallas import tpu_sc as plsc

VOCAB = 1000000
DIM = 128
B = 16384
L = 50
N = B * L

_info = plsc.get_sparse_core_info()
NC = _info.num_cores
NS = _info.num_subcores
NW = NC * NS
CHUNK = 128
PER_W = N // NW
STEPS = PER_W // CHUNK
IDX_ROWS = PER_W // CHUNK


def _sc_body(x_hbm, e_hbm, node_hbm, edge_hbm, out_hbm,
             xidx, eidx, nbuf, ebuf, sem1, sem2):
    wid = lax.axis_index("s") * NC + lax.axis_index("c")
    pltpu.sync_copy(x_hbm.at[pl.ds(wid * IDX_ROWS, IDX_ROWS)], xidx)
    pltpu.sync_copy(e_hbm.at[pl.ds(wid * IDX_ROWS, IDX_ROWS)], eidx)

    def step(i, _):
        cp1 = pltpu.async_copy(node_hbm.at[xidx.at[i]], nbuf, sem1)
        cp2 = pltpu.async_copy(edge_hbm.at[eidx.at[i]], ebuf, sem2)
        cp1.wait()
        cp2.wait()

        def add_row(r, _):
            for sl in range(DIM // 16):
                s = pl.ds(sl * 16, 16)
                nbuf[r, s] = nbuf[r, s] + ebuf[r, s]
            return 0

        lax.fori_loop(0, CHUNK, add_row, 0)
        pltpu.sync_copy(nbuf, out_hbm.at[pl.ds((wid * IDX_ROWS + i) * CHUNK, CHUNK)])
        return 0

    lax.fori_loop(0, STEPS, step, 0)


@jax.jit
def _sc_gather_add(xf, ef, node_table, edge_table):
    mesh = plsc.VectorSubcoreMesh(core_axis_name="c", subcore_axis_name="s")
    k = pl.kernel(
        _sc_body,
        out_type=jax.ShapeDtypeStruct((N, DIM), jnp.float32),
        mesh=mesh,
        scratch_types=[
            pltpu.VMEM((IDX_ROWS, CHUNK), jnp.int32),
            pltpu.VMEM((IDX_ROWS, CHUNK), jnp.int32),
            pltpu.VMEM((CHUNK, DIM), jnp.float32),
            pltpu.VMEM((CHUNK, DIM), jnp.float32),
            pltpu.SemaphoreType.DMA,
            pltpu.SemaphoreType.DMA,
        ],
    )
    return k(xf, ef, node_table, edge_table)


def _leaky(v):
    return jnp.where(v >= 0, v, 0.1 * v)


def _mlp_body(h_ref, w1_ref, b1_ref, w2_ref, b2_ref, w3_ref, b3_ref,
              w4_ref, b4_ref, out_ref):
    h = h_ref[...]
    h = _leaky(jnp.dot(h, w1_ref[...], preferred_element_type=jnp.float32) + b1_ref[...])
    h = _leaky(jnp.dot(h, w2_ref[...], preferred_element_type=jnp.float32) + b2_ref[...])
    h = _leaky(jnp.dot(h, w3_ref[...], preferred_element_type=jnp.float32) + b3_ref[...])
    out_ref[...] = jnp.sum(h * w4_ref[...], axis=1, keepdims=True) + b4_ref[...]


def _mlp(h, w1t, b1, w2t, b2, w3t, b3, w4, b4, rows_per_block=2048):
    grid = (N // rows_per_block,)
    wspec = pl.BlockSpec((DIM, DIM), lambda i: (0, 0))
    bspec = pl.BlockSpec((1, DIM), lambda i: (0, 0))
    return pl.pallas_call(
        _mlp_body,
        grid=grid,
        in_specs=[
            pl.BlockSpec((rows_per_block, DIM), lambda i: (i, 0)),
            wspec, bspec, wspec, bspec, wspec, bspec,
            bspec, pl.BlockSpec((1, 1), lambda i: (0, 0)),
        ],
        out_specs=pl.BlockSpec((rows_per_block, 1), lambda i: (i, 0)),
        out_shape=jax.ShapeDtypeStruct((N, 1), jnp.float32),
    )(h, w1t, b1, w2t, b2, w3t, b3, w4, b4)


def kernel(x, edge_attr, node_table, edge_table, W1, b1, W2, b2, W3, b3, W4, b4):
    xf = x.astype(jnp.int32).reshape(N // CHUNK, CHUNK)
    ef = edge_attr.astype(jnp.int32).reshape(N // CHUNK, CHUNK)
    h = _sc_gather_add(xf, ef, node_table, edge_table)
    logits = _mlp(h, W1.T, b1.reshape(1, DIM), W2.T, b2.reshape(1, DIM),
                  W3.T, b3.reshape(1, DIM), W4, b4.reshape(1, 1))
    return logits.reshape(B, L, 1)


# speedup vs baseline: 1.7296x; 1.7296x over previous
import functools

import jax
import jax.numpy as jnp
from jax import lax
from jax.experimental import pallas as pl
from jax.experimental.pallas import tpu as pltpu
from jax.experimental.pallas import tpu_sc as plsc

VOCAB = 1000000
DIM = 128
B = 16384
L = 50
N = B * L

NC = 2
NS = 16
NW = NC * NS
CHUNK = 128
PER_W = N // NW
STEPS = PER_W // CHUNK
IDX_ROWS = PER_W // CHUNK


def _sc_body(x_hbm, e_hbm, node_hbm, edge_hbm, out_hbm,
             xidx, eidx, nbuf, ebuf, sem1, sem2):
    wid = lax.axis_index("s") * NC + lax.axis_index("c")
    pltpu.sync_copy(x_hbm.at[pl.ds(wid * IDX_ROWS, IDX_ROWS)], xidx)
    pltpu.sync_copy(e_hbm.at[pl.ds(wid * IDX_ROWS, IDX_ROWS)], eidx)

    def step(i, _):
        cp1 = pltpu.async_copy(node_hbm.at[xidx.at[i]], nbuf, sem1)
        cp2 = pltpu.async_copy(edge_hbm.at[eidx.at[i]], ebuf, sem2)
        cp1.wait()
        cp2.wait()

        def add_row(r, _):
            for sl in range(DIM // 16):
                s = pl.ds(sl * 16, 16)
                nbuf[r, s] = nbuf[r, s] + ebuf[r, s]
            return 0

        lax.fori_loop(0, CHUNK, add_row, 0)
        pltpu.sync_copy(nbuf, out_hbm.at[pl.ds((wid * IDX_ROWS + i) * CHUNK, CHUNK)])
        return 0

    lax.fori_loop(0, STEPS, step, 0)


@jax.jit
def _sc_gather_add(xf, ef, node_table, edge_table):
    mesh = plsc.VectorSubcoreMesh(core_axis_name="c", subcore_axis_name="s")
    k = pl.kernel(
        _sc_body,
        out_type=jax.ShapeDtypeStruct((N, DIM), jnp.float32),
        mesh=mesh,
        scratch_types=[
            pltpu.VMEM((IDX_ROWS, CHUNK), jnp.int32),
            pltpu.VMEM((IDX_ROWS, CHUNK), jnp.int32),
            pltpu.VMEM((CHUNK, DIM), jnp.float32),
            pltpu.VMEM((CHUNK, DIM), jnp.float32),
            pltpu.SemaphoreType.DMA,
            pltpu.SemaphoreType.DMA,
        ],
    )
    return k(xf, ef, node_table, edge_table)


def _leaky(v):
    return jnp.where(v >= 0, v, 0.1 * v)


def _mlp_body(h_ref, w1_ref, b1_ref, w2_ref, b2_ref, w3_ref, b3_ref,
              w4_ref, b4_ref, out_ref):
    h = h_ref[...]
    h = _leaky(jnp.dot(h, w1_ref[...], preferred_element_type=jnp.float32) + b1_ref[...])
    h = _leaky(jnp.dot(h, w2_ref[...], preferred_element_type=jnp.float32) + b2_ref[...])
    h = _leaky(jnp.dot(h, w3_ref[...], preferred_element_type=jnp.float32) + b3_ref[...])
    out_ref[...] = jnp.sum(h * w4_ref[...], axis=1, keepdims=True) + b4_ref[...]


def _mlp(h, w1t, b1, w2t, b2, w3t, b3, w4, b4, rows_per_block=2048):
    grid = (N // rows_per_block,)
    wspec = pl.BlockSpec((DIM, DIM), lambda i: (0, 0))
    bspec = pl.BlockSpec((1, DIM), lambda i: (0, 0))
    return pl.pallas_call(
        _mlp_body,
        grid=grid,
        in_specs=[
            pl.BlockSpec((rows_per_block, DIM), lambda i: (i, 0)),
            wspec, bspec, wspec, bspec, wspec, bspec,
            bspec, pl.BlockSpec((1, 1), lambda i: (0, 0)),
        ],
        out_specs=pl.BlockSpec((rows_per_block, 1), lambda i: (i, 0)),
        out_shape=jax.ShapeDtypeStruct((N, 1), jnp.float32),
    )(h, w1t, b1, w2t, b2, w3t, b3, w4, b4)


def kernel(x, edge_attr, node_table, edge_table, W1, b1, W2, b2, W3, b3, W4, b4):
    xf = x.astype(jnp.int32).reshape(N // CHUNK, CHUNK)
    ef = edge_attr.astype(jnp.int32).reshape(N // CHUNK, CHUNK)
    h = _sc_gather_add(xf, ef, node_table, edge_table)
    logits = _mlp(h, W1.T, b1.reshape(1, DIM), W2.T, b2.reshape(1, DIM),
                  W3.T, b3.reshape(1, DIM), W4, b4.reshape(1, 1))
    return logits.reshape(B, L, 1)


# speedup vs baseline: 2.0159x; 1.1655x over previous
import functools

import jax
import jax.numpy as jnp
from jax import lax
from jax.experimental import pallas as pl
from jax.experimental.pallas import tpu as pltpu
from jax.experimental.pallas import tpu_sc as plsc

VOCAB = 1000000
DIM = 128
B = 16384
L = 50
N = B * L

NC = 2
NS = 16
NW = NC * NS
CHUNK = 128
PER_W = N // NW
STEPS = PER_W // CHUNK
IDX_ROWS = PER_W // CHUNK


def _sc_body(x_hbm, e_hbm, node_hbm, edge_hbm, out_hbm,
             xidx, eidx, nbuf, ebuf, sem1, sem2):
    wid = lax.axis_index("s") * NC + lax.axis_index("c")
    pltpu.sync_copy(x_hbm.at[pl.ds(wid * IDX_ROWS, IDX_ROWS)], xidx)
    pltpu.sync_copy(e_hbm.at[pl.ds(wid * IDX_ROWS, IDX_ROWS)], eidx)

    def step(i, _):
        cp1 = pltpu.async_copy(node_hbm.at[xidx.at[i]], nbuf, sem1)
        cp2 = pltpu.async_copy(edge_hbm.at[eidx.at[i]], ebuf, sem2)
        cp1.wait()
        cp2.wait()

        def add_row(r, _):
            for sl in range(DIM // 16):
                s = pl.ds(sl * 16, 16)
                nbuf[r, s] = nbuf[r, s] + ebuf[r, s]
            return 0

        lax.fori_loop(0, CHUNK, add_row, 0)
        pltpu.sync_copy(nbuf, out_hbm.at[pl.ds((wid * IDX_ROWS + i) * CHUNK, CHUNK)])
        return 0

    lax.fori_loop(0, STEPS, step, 0)


@jax.jit
def _sc_gather_add(xf, ef, node_table, edge_table):
    mesh = plsc.VectorSubcoreMesh(core_axis_name="c", subcore_axis_name="s",
                                  num_cores=NC)
    k = pl.kernel(
        _sc_body,
        out_type=jax.ShapeDtypeStruct((N, DIM), jnp.float32),
        mesh=mesh,
        scratch_types=[
            pltpu.VMEM((IDX_ROWS, CHUNK), jnp.int32),
            pltpu.VMEM((IDX_ROWS, CHUNK), jnp.int32),
            pltpu.VMEM((CHUNK, DIM), jnp.float32),
            pltpu.VMEM((CHUNK, DIM), jnp.float32),
            pltpu.SemaphoreType.DMA,
            pltpu.SemaphoreType.DMA,
        ],
    )
    return k(xf, ef, node_table, edge_table)


def _leaky(v):
    return jnp.where(v >= 0, v, 0.1 * v)


def _mlp_body(h_ref, w1_ref, b1_ref, w2_ref, b2_ref, w3_ref, b3_ref,
              w4_ref, b4_ref, out_ref):
    h = h_ref[...]
    h = _leaky(jnp.dot(h, w1_ref[...], preferred_element_type=jnp.float32) + b1_ref[...])
    h = _leaky(jnp.dot(h, w2_ref[...], preferred_element_type=jnp.float32) + b2_ref[...])
    h = _leaky(jnp.dot(h, w3_ref[...], preferred_element_type=jnp.float32) + b3_ref[...])
    v = jnp.sum(h * w4_ref[...], axis=1) + b4_ref[0, 0]
    out_ref[...] = v.reshape(out_ref.shape)


def _mlp(h, w1t, b1, w2t, b2, w3t, b3, w4, b4, rows_per_block=2048):
    grid = (N // rows_per_block,)
    wspec = pl.BlockSpec((DIM, DIM), lambda i: (0, 0))
    bspec = pl.BlockSpec((1, DIM), lambda i: (0, 0))
    return pl.pallas_call(
        _mlp_body,
        grid=grid,
        in_specs=[
            pl.BlockSpec((rows_per_block, DIM), lambda i: (i, 0)),
            wspec, bspec, wspec, bspec, wspec, bspec,
            bspec, pl.BlockSpec((1, 1), lambda i: (0, 0)),
        ],
        out_specs=pl.BlockSpec((rows_per_block // DIM, DIM), lambda i: (i, 0)),
        out_shape=jax.ShapeDtypeStruct((N // DIM, DIM), jnp.float32),
    )(h, w1t, b1, w2t, b2, w3t, b3, w4, b4)


def kernel(x, edge_attr, node_table, edge_table, W1, b1, W2, b2, W3, b3, W4, b4):
    xf = x.astype(jnp.int32).reshape(N // CHUNK, CHUNK)
    ef = edge_attr.astype(jnp.int32).reshape(N // CHUNK, CHUNK)
    h = _sc_gather_add(xf, ef, node_table, edge_table)
    logits = _mlp(h, W1.T, b1.reshape(1, DIM), W2.T, b2.reshape(1, DIM),
                  W3.T, b3.reshape(1, DIM), W4, b4.reshape(1, 1))
    return logits.reshape(B, L, 1)


# speedup vs baseline: 2.7177x; 1.3481x over previous
import functools

import jax
import jax.numpy as jnp
from jax import lax
from jax.experimental import pallas as pl
from jax.experimental.pallas import tpu as pltpu
from jax.experimental.pallas import tpu_sc as plsc

VOCAB = 1000000
DIM = 128
B = 16384
L = 50
N = B * L

NC = 2
NS = 16
NW = NC * NS
CHUNK = 128
PER_W = N // NW
STEPS = PER_W // CHUNK
IDX_ROWS = PER_W // CHUNK


def _sc_body(x_hbm, e_hbm, node_hbm, edge_hbm, out_hbm,
             xidx, eidx, nbuf, ebuf, gn0, ge0, gn1, ge1, ss0, ss1):
    wid = lax.axis_index("s") * NC + lax.axis_index("c")
    gn = (gn0, gn1)
    ge = (ge0, ge1)
    ss = (ss0, ss1)
    pltpu.sync_copy(x_hbm.at[pl.ds(wid * IDX_ROWS, IDX_ROWS)], xidx)
    pltpu.sync_copy(e_hbm.at[pl.ds(wid * IDX_ROWS, IDX_ROWS)], eidx)

    def start_gather(i, b):
        pltpu.async_copy(node_hbm.at[xidx.at[i]], nbuf.at[b], gn[b])
        pltpu.async_copy(edge_hbm.at[eidx.at[i]], ebuf.at[b], ge[b])

    def wait_gather(i, b):
        pltpu.make_async_copy(node_hbm.at[xidx.at[i]], nbuf.at[b], gn[b]).wait()
        pltpu.make_async_copy(edge_hbm.at[eidx.at[i]], ebuf.at[b], ge[b]).wait()

    def out_slice(i):
        return out_hbm.at[pl.ds((wid * IDX_ROWS + i) * CHUNK, CHUNK)]

    def add_and_store(i, b):
        def add_row(r, _):
            for sl in range(DIM // 16):
                s = pl.ds(sl * 16, 16)
                plsc.addupdate(nbuf.at[b, r, s], ebuf[b, r, s])
            return 0

        lax.fori_loop(0, CHUNK, add_row, 0)
        pltpu.async_copy(nbuf.at[b], out_slice(i), ss[b])

    start_gather(0, 0)
    start_gather(1, 1)
    wait_gather(0, 0)
    add_and_store(0, 0)

    def step(i, b):
        nb = 1 - b
        pltpu.make_async_copy(nbuf.at[nb], out_slice(i - 1), ss[nb]).wait()
        start_gather(i + 1, nb)
        wait_gather(i, b)
        add_and_store(i, b)

    def pair(j, _):
        step(1 + 2 * j, 1)
        step(2 + 2 * j, 0)
        return 0

    lax.fori_loop(0, (STEPS - 2) // 2, pair, 0)

    b_last = (STEPS - 1) % 2
    wait_gather(STEPS - 1, b_last)
    add_and_store(STEPS - 1, b_last)
    pltpu.make_async_copy(nbuf.at[1 - b_last], out_slice(STEPS - 2), ss[1 - b_last]).wait()
    pltpu.make_async_copy(nbuf.at[b_last], out_slice(STEPS - 1), ss[b_last]).wait()


@jax.jit
def _sc_gather_add(xf, ef, node_table, edge_table):
    mesh = plsc.VectorSubcoreMesh(core_axis_name="c", subcore_axis_name="s",
                                  num_cores=NC)
    k = pl.kernel(
        _sc_body,
        out_type=jax.ShapeDtypeStruct((N, DIM), jnp.float32),
        mesh=mesh,
        scratch_types=[
            pltpu.VMEM((IDX_ROWS, CHUNK), jnp.int32),
            pltpu.VMEM((IDX_ROWS, CHUNK), jnp.int32),
            pltpu.VMEM((2, CHUNK, DIM), jnp.float32),
            pltpu.VMEM((2, CHUNK, DIM), jnp.float32),
            pltpu.SemaphoreType.DMA,
            pltpu.SemaphoreType.DMA,
            pltpu.SemaphoreType.DMA,
            pltpu.SemaphoreType.DMA,
            pltpu.SemaphoreType.DMA,
            pltpu.SemaphoreType.DMA,
        ],
    )
    return k(xf, ef, node_table, edge_table)


def _leaky(v):
    return jnp.where(v >= 0, v, 0.1 * v)


def _mlp_body(h_ref, w1_ref, b1_ref, w2_ref, b2_ref, w3_ref, b3_ref,
              w4_ref, b4_ref, out_ref):
    h = h_ref[...]
    h = _leaky(jnp.dot(h, w1_ref[...], preferred_element_type=jnp.float32) + b1_ref[...])
    h = _leaky(jnp.dot(h, w2_ref[...], preferred_element_type=jnp.float32) + b2_ref[...])
    h = _leaky(jnp.dot(h, w3_ref[...], preferred_element_type=jnp.float32) + b3_ref[...])
    v = jnp.sum(h * w4_ref[...], axis=1) + b4_ref[0, 0]
    out_ref[...] = v.reshape(out_ref.shape)


def _mlp(h, w1t, b1, w2t, b2, w3t, b3, w4, b4, rows_per_block=2048):
    grid = (N // rows_per_block,)
    wspec = pl.BlockSpec((DIM, DIM), lambda i: (0, 0))
    bspec = pl.BlockSpec((1, DIM), lambda i: (0, 0))
    return pl.pallas_call(
        _mlp_body,
        grid=grid,
        in_specs=[
            pl.BlockSpec((rows_per_block, DIM), lambda i: (i, 0)),
            wspec, bspec, wspec, bspec, wspec, bspec,
            bspec, pl.BlockSpec((1, 1), lambda i: (0, 0)),
        ],
        out_specs=pl.BlockSpec((rows_per_block // DIM, DIM), lambda i: (i, 0)),
        out_shape=jax.ShapeDtypeStruct((N // DIM, DIM), jnp.float32),
    )(h, w1t, b1, w2t, b2, w3t, b3, w4, b4)


def kernel(x, edge_attr, node_table, edge_table, W1, b1, W2, b2, W3, b3, W4, b4):
    xf = x.astype(jnp.int32).reshape(N // CHUNK, CHUNK)
    ef = edge_attr.astype(jnp.int32).reshape(N // CHUNK, CHUNK)
    h = _sc_gather_add(xf, ef, node_table, edge_table)
    logits = _mlp(h, W1.T, b1.reshape(1, DIM), W2.T, b2.reshape(1, DIM),
                  W3.T, b3.reshape(1, DIM), W4, b4.reshape(1, 1))
    return logits.reshape(B, L, 1)


# speedup vs baseline: 2.7363x; 1.0069x over previous
import functools

import jax
import jax.numpy as jnp
from jax import lax
from jax.experimental import pallas as pl
from jax.experimental.pallas import tpu as pltpu
from jax.experimental.pallas import tpu_sc as plsc

VOCAB = 1000000
DIM = 128
B = 16384
L = 50
N = B * L

NC = 2
NS = 16
NW = NC * NS
CHUNK = 64
PER_W = N // NW
STEPS = PER_W // CHUNK
IDX_ROWS = PER_W // CHUNK
NSLOT = 3


def _sc_body(x_hbm, e_hbm, node_hbm, edge_hbm, out_hbm,
             xidx, eidx, nbuf, ebuf, *sems):
    wid = lax.axis_index("s") * NC + lax.axis_index("c")
    gn = sems[0:NSLOT]
    ge = sems[NSLOT:2 * NSLOT]
    ss = sems[2 * NSLOT:3 * NSLOT]
    pltpu.sync_copy(x_hbm.at[pl.ds(wid * PER_W, PER_W)], xidx)
    pltpu.sync_copy(e_hbm.at[pl.ds(wid * PER_W, PER_W)], eidx)

    def start_gather(i, b):
        s = pl.ds(i * CHUNK, CHUNK)
        pltpu.async_copy(node_hbm.at[xidx.at[s]], nbuf.at[b], gn[b])
        pltpu.async_copy(edge_hbm.at[eidx.at[s]], ebuf.at[b], ge[b])

    def wait_gather(i, b):
        s = pl.ds(i * CHUNK, CHUNK)
        pltpu.make_async_copy(node_hbm.at[xidx.at[s]], nbuf.at[b], gn[b]).wait()
        pltpu.make_async_copy(edge_hbm.at[eidx.at[s]], ebuf.at[b], ge[b]).wait()

    def out_slice(i):
        return out_hbm.at[pl.ds((wid * IDX_ROWS + i) * CHUNK, CHUNK)]

    def wait_store(i, b):
        pltpu.make_async_copy(nbuf.at[b], out_slice(i), ss[b]).wait()

    def add_and_store(i, b):
        def add_row(r, _):
            for sl in range(DIM // 16):
                s = pl.ds(sl * 16, 16)
                plsc.addupdate(nbuf.at[b, r, s], ebuf[b, r, s])
            return 0

        lax.fori_loop(0, CHUNK, add_row, 0)
        pltpu.async_copy(nbuf.at[b], out_slice(i), ss[b])

    def do_step(i, b):
        nb = (b + 2) % NSLOT
        wait_store(i - 1, nb)
        start_gather(i + 2, nb)
        wait_gather(i, b)
        add_and_store(i, b)

    start_gather(0, 0)
    start_gather(1, 1)
    start_gather(2, 2)
    wait_gather(0, 0)
    add_and_store(0, 0)
    wait_store(0, 0)
    start_gather(3, 0)
    wait_gather(1, 1)
    add_and_store(1, 1)

    def triple(j, _):
        i = 2 + 3 * j
        do_step(i, 2)
        do_step(i + 1, 0)
        do_step(i + 2, 1)
        return 0

    lax.fori_loop(0, (STEPS - 4) // 3, triple, 0)

    wait_gather(STEPS - 2, (STEPS - 2) % NSLOT)
    add_and_store(STEPS - 2, (STEPS - 2) % NSLOT)
    wait_gather(STEPS - 1, (STEPS - 1) % NSLOT)
    add_and_store(STEPS - 1, (STEPS - 1) % NSLOT)
    for k in range(3):
        wait_store(STEPS - 3 + k, (STEPS - 3 + k) % NSLOT)


@jax.jit
def _sc_gather_add(xf, ef, node_table, edge_table):
    mesh = plsc.VectorSubcoreMesh(core_axis_name="c", subcore_axis_name="s",
                                  num_cores=NC)
    k = pl.kernel(
        _sc_body,
        out_type=jax.ShapeDtypeStruct((N, DIM), jnp.float32),
        mesh=mesh,
        scratch_types=[
            pltpu.VMEM((PER_W,), jnp.int32),
            pltpu.VMEM((PER_W,), jnp.int32),
            pltpu.VMEM((NSLOT, CHUNK, DIM), jnp.float32),
            pltpu.VMEM((NSLOT, CHUNK, DIM), jnp.float32),
        ] + [pltpu.SemaphoreType.DMA] * (3 * NSLOT),
    )
    return k(xf, ef, node_table, edge_table)


def _leaky(v):
    return jnp.where(v >= 0, v, 0.1 * v)


def _mlp_body(h_ref, w1_ref, b1_ref, w2_ref, b2_ref, w3_ref, b3_ref,
              w4_ref, b4_ref, out_ref):
    h = h_ref[...]
    h = _leaky(jnp.dot(h, w1_ref[...], preferred_element_type=jnp.float32) + b1_ref[...])
    h = _leaky(jnp.dot(h, w2_ref[...], preferred_element_type=jnp.float32) + b2_ref[...])
    h = _leaky(jnp.dot(h, w3_ref[...], preferred_element_type=jnp.float32) + b3_ref[...])
    v = jnp.sum(h * w4_ref[...], axis=1) + b4_ref[0, 0]
    out_ref[...] = v.reshape(out_ref.shape)


def _mlp(h, w1t, b1, w2t, b2, w3t, b3, w4, b4, rows_per_block=2048):
    grid = (N // rows_per_block,)
    wspec = pl.BlockSpec((DIM, DIM), lambda i: (0, 0))
    bspec = pl.BlockSpec((1, DIM), lambda i: (0, 0))
    return pl.pallas_call(
        _mlp_body,
        grid=grid,
        in_specs=[
            pl.BlockSpec((rows_per_block, DIM), lambda i: (i, 0)),
            wspec, bspec, wspec, bspec, wspec, bspec,
            bspec, pl.BlockSpec((1, 1), lambda i: (0, 0)),
        ],
        out_specs=pl.BlockSpec((rows_per_block // DIM, DIM), lambda i: (i, 0)),
        out_shape=jax.ShapeDtypeStruct((N // DIM, DIM), jnp.float32),
    )(h, w1t, b1, w2t, b2, w3t, b3, w4, b4)


def kernel(x, edge_attr, node_table, edge_table, W1, b1, W2, b2, W3, b3, W4, b4):
    xf = x.astype(jnp.int32).reshape(N)
    ef = edge_attr.astype(jnp.int32).reshape(N)
    h = _sc_gather_add(xf, ef, node_table, edge_table)
    logits = _mlp(h, W1.T, b1.reshape(1, DIM), W2.T, b2.reshape(1, DIM),
                  W3.T, b3.reshape(1, DIM), W4, b4.reshape(1, 1))
    return logits.reshape(B, L, 1)


# speedup vs baseline: 3.2566x; 1.1901x over previous
import functools

import jax
import jax.numpy as jnp
from jax import lax
from jax.experimental import pallas as pl
from jax.experimental.pallas import tpu as pltpu
from jax.experimental.pallas import tpu_sc as plsc

VOCAB = 1000000
DIM = 128
B = 16384
L = 50
N = B * L

NC = 2
NS = 16
NW = NC * NS
CHUNK = 64
PER_W = N // NW
STEPS = PER_W // CHUNK
IDX_ROWS = PER_W // CHUNK
NSLOT = 3


def _sc_body(x_hbm, e_hbm, node_hbm, edge_hbm, out_hbm,
             xidx, eidx, nbuf, ebuf, *sems):
    wid = lax.axis_index("s") * NC + lax.axis_index("c")
    gn = sems[0:NSLOT]
    ge = sems[NSLOT:2 * NSLOT]
    ss = sems[2 * NSLOT:3 * NSLOT]
    pltpu.sync_copy(x_hbm.at[pl.ds(wid * PER_W, PER_W)], xidx)
    pltpu.sync_copy(e_hbm.at[pl.ds(wid * PER_W, PER_W)], eidx)

    def start_gather(i, b):
        s = pl.ds(i * CHUNK, CHUNK)
        pltpu.async_copy(node_hbm.at[xidx.at[s]], nbuf.at[b], gn[b])
        pltpu.async_copy(edge_hbm.at[eidx.at[s]], ebuf.at[b], ge[b])

    def wait_gather(i, b):
        s = pl.ds(i * CHUNK, CHUNK)
        pltpu.make_async_copy(node_hbm.at[xidx.at[s]], nbuf.at[b], gn[b]).wait()
        pltpu.make_async_copy(edge_hbm.at[eidx.at[s]], ebuf.at[b], ge[b]).wait()

    def out_slice(i):
        return out_hbm.at[pl.ds((wid * IDX_ROWS + i) * CHUNK, CHUNK)]

    def wait_store(i, b):
        pltpu.make_async_copy(nbuf.at[b], out_slice(i), ss[b]).wait()

    def add_and_store(i, b):
        def add_row(r, _):
            for sl in range(DIM // 16):
                s = pl.ds(sl * 16, 16)
                plsc.addupdate(nbuf.at[b, r, s], ebuf[b, r, s])
            return 0

        lax.fori_loop(0, CHUNK, add_row, 0)
        pltpu.async_copy(nbuf.at[b], out_slice(i), ss[b])

    def do_step(i, b):
        nb = (b + 2) % NSLOT
        wait_store(i - 1, nb)
        start_gather(i + 2, nb)
        wait_gather(i, b)
        add_and_store(i, b)

    start_gather(0, 0)
    start_gather(1, 1)
    start_gather(2, 2)
    wait_gather(0, 0)
    add_and_store(0, 0)
    wait_store(0, 0)
    start_gather(3, 0)
    wait_gather(1, 1)
    add_and_store(1, 1)

    def triple(j, _):
        i = 2 + 3 * j
        do_step(i, 2)
        do_step(i + 1, 0)
        do_step(i + 2, 1)
        return 0

    lax.fori_loop(0, (STEPS - 4) // 3, triple, 0)

    wait_gather(STEPS - 2, (STEPS - 2) % NSLOT)
    add_and_store(STEPS - 2, (STEPS - 2) % NSLOT)
    wait_gather(STEPS - 1, (STEPS - 1) % NSLOT)
    add_and_store(STEPS - 1, (STEPS - 1) % NSLOT)
    for k in range(3):
        wait_store(STEPS - 3 + k, (STEPS - 3 + k) % NSLOT)


@jax.jit
def _sc_gather_add(xf, ef, node_table, edge_table):
    mesh = plsc.VectorSubcoreMesh(core_axis_name="c", subcore_axis_name="s",
                                  num_cores=NC)
    k = pl.kernel(
        _sc_body,
        out_type=jax.ShapeDtypeStruct((N, DIM), jnp.float32),
        mesh=mesh,
        scratch_types=[
            pltpu.VMEM((PER_W,), jnp.int32),
            pltpu.VMEM((PER_W,), jnp.int32),
            pltpu.VMEM((NSLOT, CHUNK, DIM), jnp.float32),
            pltpu.VMEM((NSLOT, CHUNK, DIM), jnp.float32),
        ] + [pltpu.SemaphoreType.DMA] * (3 * NSLOT),
    )
    return k(xf, ef, node_table, edge_table)


def _leaky(v):
    return jnp.maximum(v, 0.1 * v)


def _mlp_body(h_ref, w1_ref, b1_ref, w2_ref, b2_ref, w3_ref, b3_ref,
              w4_ref, b4_ref, out_ref):
    def lin(v, w_ref, b_ref):
        return jnp.dot(v, w_ref[...], preferred_element_type=jnp.float32) + b_ref[...]

    h = _leaky(lin(h_ref[...], w1_ref, b1_ref))
    h = _leaky(lin(h, w2_ref, b2_ref))
    h = _leaky(lin(h, w3_ref, b3_ref))
    v = jnp.sum(h * w4_ref[...], axis=1) + b4_ref[0, 0]
    out_ref[...] = v.reshape(out_ref.shape)


def _mlp(h, w1t, b1, w2t, b2, w3t, b3, w4, b4, rows_per_block=8192):
    grid = (N // rows_per_block,)
    wspec = pl.BlockSpec((DIM, DIM), lambda i: (0, 0))
    bspec = pl.BlockSpec((1, DIM), lambda i: (0, 0))
    return pl.pallas_call(
        _mlp_body,
        grid=grid,
        in_specs=[
            pl.BlockSpec((rows_per_block, DIM), lambda i: (i, 0)),
            wspec, bspec, wspec, bspec, wspec, bspec,
            bspec,
            pl.BlockSpec((1, 1), lambda i: (0, 0)),
        ],
        out_specs=pl.BlockSpec((rows_per_block // DIM, DIM), lambda i: (i, 0)),
        out_shape=jax.ShapeDtypeStruct((N // DIM, DIM), jnp.float32),
    )(h, w1t, b1, w2t, b2, w3t, b3, w4, b4)


def kernel(x, edge_attr, node_table, edge_table, W1, b1, W2, b2, W3, b3, W4, b4):
    xf = x.astype(jnp.int32).reshape(N)
    ef = edge_attr.astype(jnp.int32).reshape(N)
    h = _sc_gather_add(xf, ef, node_table, edge_table)
    logits = _mlp(h, W1.T, b1.reshape(1, DIM), W2.T, b2.reshape(1, DIM),
                  W3.T, b3.reshape(1, DIM), W4, b4.reshape(1, 1))
    return logits.reshape(B, L, 1)


# speedup vs baseline: 3.7288x; 1.1450x over previous
import functools

import jax
import jax.numpy as jnp
from jax import lax
from jax.experimental import pallas as pl
from jax.experimental.pallas import tpu as pltpu
from jax.experimental.pallas import tpu_sc as plsc

VOCAB = 1000000
DIM = 128
B = 16384
L = 50
N = B * L

NC = 2
NS = 16
NW = NC * NS
CHUNK = 64
NSLOT = 3
NPART = 4
NP = N // NPART
PER_W = NP // NW
STEPS = PER_W // CHUNK


def _make_sc_body(part):
    base = part * NP

    def _sc_body(x_hbm, e_hbm, node_hbm, edge_hbm, out_hbm,
                 xidx, eidx, nbuf, ebuf, *sems):
        wid = lax.axis_index("s") * NC + lax.axis_index("c")
        gn = sems[0:NSLOT]
        ge = sems[NSLOT:2 * NSLOT]
        ss = sems[2 * NSLOT:3 * NSLOT]
        pltpu.sync_copy(x_hbm.at[pl.ds(base + wid * PER_W, PER_W)], xidx)
        pltpu.sync_copy(e_hbm.at[pl.ds(base + wid * PER_W, PER_W)], eidx)

        def start_gather(i, b):
            s = pl.ds(i * CHUNK, CHUNK)
            pltpu.async_copy(node_hbm.at[xidx.at[s]], nbuf.at[b], gn[b])
            pltpu.async_copy(edge_hbm.at[eidx.at[s]], ebuf.at[b], ge[b])

        def wait_gather(i, b):
            s = pl.ds(i * CHUNK, CHUNK)
            pltpu.make_async_copy(node_hbm.at[xidx.at[s]], nbuf.at[b], gn[b]).wait()
            pltpu.make_async_copy(edge_hbm.at[eidx.at[s]], ebuf.at[b], ge[b]).wait()

        def out_slice(i):
            return out_hbm.at[pl.ds((wid * STEPS + i) * CHUNK, CHUNK)]

        def wait_store(i, b):
            pltpu.make_async_copy(nbuf.at[b], out_slice(i), ss[b]).wait()

        def add_and_store(i, b):
            def add_row(r, _):
                for sl in range(DIM // 16):
                    s = pl.ds(sl * 16, 16)
                    plsc.addupdate(nbuf.at[b, r, s], ebuf[b, r, s])
                return 0

            lax.fori_loop(0, CHUNK, add_row, 0)
            pltpu.async_copy(nbuf.at[b], out_slice(i), ss[b])

        def do_step(i, b):
            nb = (b + 2) % NSLOT
            wait_store(i - 1, nb)
            start_gather(i + 2, nb)
            wait_gather(i, b)
            add_and_store(i, b)

        start_gather(0, 0)
        start_gather(1, 1)
        start_gather(2, 2)
        wait_gather(0, 0)
        add_and_store(0, 0)
        wait_store(0, 0)
        start_gather(3, 0)
        wait_gather(1, 1)
        add_and_store(1, 1)

        def triple(j, _):
            i = 2 + 3 * j
            do_step(i, 2)
            do_step(i + 1, 0)
            do_step(i + 2, 1)
            return 0

        lax.fori_loop(0, (STEPS - 4) // 3, triple, 0)

        wait_gather(STEPS - 2, (STEPS - 2) % NSLOT)
        add_and_store(STEPS - 2, (STEPS - 2) % NSLOT)
        wait_gather(STEPS - 1, (STEPS - 1) % NSLOT)
        add_and_store(STEPS - 1, (STEPS - 1) % NSLOT)
        for k in range(3):
            wait_store(STEPS - 3 + k, (STEPS - 3 + k) % NSLOT)

    return _sc_body


def _sc_gather_add(part, xf, ef, node_table, edge_table):
    mesh = plsc.VectorSubcoreMesh(core_axis_name="c", subcore_axis_name="s",
                                  num_cores=NC)
    k = pl.kernel(
        _make_sc_body(part),
        out_type=jax.ShapeDtypeStruct((NP, DIM), jnp.float32),
        mesh=mesh,
        scratch_types=[
            pltpu.VMEM((PER_W,), jnp.int32),
            pltpu.VMEM((PER_W,), jnp.int32),
            pltpu.VMEM((NSLOT, CHUNK, DIM), jnp.float32),
            pltpu.VMEM((NSLOT, CHUNK, DIM), jnp.float32),
        ] + [pltpu.SemaphoreType.DMA] * (3 * NSLOT),
    )
    return k(xf, ef, node_table, edge_table)


def _leaky(v):
    return jnp.maximum(v, 0.1 * v)


def _mlp_body(h_ref, w1_ref, b1_ref, w2_ref, b2_ref, w3_ref, b3_ref,
              w4_ref, b4_ref, out_ref):
    def lin(v, w_ref, b_ref):
        return jnp.dot(v, w_ref[...], preferred_element_type=jnp.float32) + b_ref[...]

    h = _leaky(lin(h_ref[...], w1_ref, b1_ref))
    h = _leaky(lin(h, w2_ref, b2_ref))
    h = _leaky(lin(h, w3_ref, b3_ref))
    v = jnp.sum(h * w4_ref[...], axis=1) + b4_ref[0, 0]
    out_ref[...] = v.reshape(out_ref.shape)


def _mlp(h, w1t, b1, w2t, b2, w3t, b3, w4, b4, rows_per_block=8192):
    rows = h.shape[0]
    grid = (rows // rows_per_block,)
    wspec = pl.BlockSpec((DIM, DIM), lambda i: (0, 0))
    bspec = pl.BlockSpec((1, DIM), lambda i: (0, 0))
    return pl.pallas_call(
        _mlp_body,
        grid=grid,
        in_specs=[
            pl.BlockSpec((rows_per_block, DIM), lambda i: (i, 0)),
            wspec, bspec, wspec, bspec, wspec, bspec,
            bspec,
            pl.BlockSpec((1, 1), lambda i: (0, 0)),
        ],
        out_specs=pl.BlockSpec((rows_per_block // DIM, DIM), lambda i: (i, 0)),
        out_shape=jax.ShapeDtypeStruct((rows // DIM, DIM), jnp.float32),
    )(h, w1t, b1, w2t, b2, w3t, b3, w4, b4)


def kernel(x, edge_attr, node_table, edge_table, W1, b1, W2, b2, W3, b3, W4, b4):
    xf = x.astype(jnp.int32).reshape(N)
    ef = edge_attr.astype(jnp.int32).reshape(N)
    parts = []
    for k in range(NPART):
        h = _sc_gather_add(k, xf, ef, node_table, edge_table)
        parts.append(_mlp(h, W1.T, b1.reshape(1, DIM), W2.T, b2.reshape(1, DIM),
                          W3.T, b3.reshape(1, DIM), W4, b4.reshape(1, 1)))
    logits = jnp.concatenate(parts, axis=0)
    return logits.reshape(B, L, 1)
